# Initial kernel scaffold; baseline (speedup 1.0000x reference)
import functools

import jax
import jax.numpy as jnp
from jax import lax
from jax.experimental import pallas as pl
from jax.experimental.pallas import tpu as pltpu
from jax.experimental.pallas import tpu_sc as plsc

N = 10000
E = 160000
D = 256
NC = 2
NS = 16
DH = D // NC
B = 128
NB = 80
EPT = NB * B
EPAD = NS * EPT
N_ACC = 10016
ZSTRIPE = N_ACC // NS
WSTRIPE = N // NS
DEG_PT = EPAD // (NC * NS)
DEG_R = 640

_MESH = dict(core_axis_name="c", subcore_axis_name="s")


def _sc_deg(dstd, iid):
    @functools.partial(
        pl.kernel,
        out_type=jax.ShapeDtypeStruct((NC, DEG_R, 16), jnp.float32),
        mesh=plsc.VectorSubcoreMesh(**_MESH),
        scratch_types=[
            pltpu.VMEM((DEG_PT,), jnp.int32),
            pltpu.VMEM((DEG_R, 16), jnp.float32),
            pltpu.VMEM((5, 128), jnp.int32),
            pltpu.VMEM_SHARED((DEG_R, 16), jnp.float32),
        ],
    )
    def k(dstd_hbm, iid_hbm, degp_hbm, dstv, degloc, iidv, deg_sh):
        c = lax.axis_index("c")
        s = lax.axis_index("s")
        w = s * NC + c
        pltpu.sync_copy(dstd_hbm.at[w], dstv)
        pltpu.sync_copy(iid_hbm, iidv)
        z16 = jnp.zeros((16,), jnp.float32)
        for r in range(DEG_R):
            degloc[r, :] = z16
        pltpu.sync_copy(degloc.at[pl.ds(s * 40, 40)], deg_sh.at[pl.ds(s * 40, 40)])
        plsc.subcore_barrier()
        ones = jnp.ones((16,), jnp.float32)

        def it(i, carry):
            d = dstv[pl.ds(i * 16, 16)]
            plsc.addupdate_scatter(degloc, [d >> 4, d & 15], ones)
            return carry

        lax.fori_loop(0, DEG_PT // 16, it, 0)
        for kk in range(5):
            pltpu.sync_copy(degloc.at[pl.ds(kk * 128, 128)],
                            deg_sh.at[iidv.at[kk]], add=True)
        plsc.subcore_barrier()
        pltpu.sync_copy(deg_sh.at[pl.ds(s * 40, 40)],
                        degp_hbm.at[c].at[pl.ds(s * 40, 40)])

    return k(dstd, iid)


def _sc_agg(xs2, srcp, dstp):
    @functools.partial(
        pl.kernel,
        out_type=jax.ShapeDtypeStruct((NC, N, DH), jnp.float32),
        mesh=plsc.VectorSubcoreMesh(**_MESH),
        scratch_types=[
            pltpu.VMEM((NB, B), jnp.int32),
            pltpu.VMEM((NB, B), jnp.int32),
            pltpu.VMEM((2, B, DH), jnp.float32),
            pltpu.VMEM_SHARED((N_ACC, DH), jnp.float32),
            pltpu.SemaphoreType.DMA,
            pltpu.SemaphoreType.DMA,
        ],
    )
    def k(xs_hbm, src_hbm, dst_hbm, acc_hbm, srcv, dstv, rows, acc_sh, sem0, sem1):
        c = lax.axis_index("c")
        s = lax.axis_index("s")
        pltpu.sync_copy(src_hbm.at[s], srcv)
        pltpu.sync_copy(dst_hbm.at[s], dstv)
        z16 = jnp.zeros((16,), jnp.float32)
        for r in range(B):
            for q in range(DH // 16):
                rows[0, r, pl.ds(q * 16, 16)] = z16
        base = s * ZSTRIPE
        for q in range(4):
            pltpu.sync_copy(rows.at[0], acc_sh.at[pl.ds(base + q * B, B)])
        pltpu.sync_copy(rows.at[0, pl.ds(0, ZSTRIPE - 4 * B)],
                        acc_sh.at[pl.ds(base + 4 * B, ZSTRIPE - 4 * B)])
        plsc.subcore_barrier()

        xs_c = xs_hbm.at[c]

        def blk(jj, carry):
            j0 = jj * 2
            j1 = j0 + 1
            cp0 = pltpu.async_copy(xs_c.at[srcv.at[j0]], rows.at[0], sem0)
            cp1 = pltpu.async_copy(xs_c.at[srcv.at[j1]], rows.at[1], sem1)
            cp0.wait()
            pltpu.sync_copy(rows.at[0], acc_sh.at[dstv.at[j0]], add=True)
            cp1.wait()
            pltpu.sync_copy(rows.at[1], acc_sh.at[dstv.at[j1]], add=True)
            return carry

        lax.fori_loop(0, NB // 2, blk, 0)
        plsc.subcore_barrier()
        wb = s * WSTRIPE
        pltpu.sync_copy(acc_sh.at[pl.ds(wb, WSTRIPE)],
                        acc_hbm.at[c].at[pl.ds(wb, WSTRIPE)])

    return k(xs2, srcp, dstp)


NBLK = 1000


def _tc_dinv(degp2):
    def body(d_ref, o_ref):
        d = d_ref[...]
        o_ref[...] = lax.rsqrt(d[0:1, :] + d[1:2, :] + 1.0)

    npad = degp2.shape[1]
    blk = 1024
    return pl.pallas_call(
        body,
        grid=(npad // blk,),
        in_specs=[pl.BlockSpec((2, blk), lambda i: (0, i))],
        out_specs=pl.BlockSpec((1, blk), lambda i: (0, i)),
        out_shape=jax.ShapeDtypeStruct((1, npad), jnp.float32),
    )(degp2)


def _tc_first(feats, W, dinv):
    def body(x_ref, w_ref, dv_ref, o_ref):
        x = x_ref[...]
        o_ref[...] = jnp.dot(x, w_ref[...],
                             preferred_element_type=jnp.float32) * dv_ref[...]

    return pl.pallas_call(
        body,
        grid=(NC, N // NBLK),
        in_specs=[
            pl.BlockSpec((NBLK, D), lambda c, i: (i, 0)),
            pl.BlockSpec((D, DH), lambda c, i: (0, c)),
            pl.BlockSpec((NBLK, 1), lambda c, i: (i, 0)),
        ],
        out_specs=pl.BlockSpec((None, NBLK, DH), lambda c, i: (c, i, 0)),
        out_shape=jax.ShapeDtypeStruct((NC, N, DH), jnp.float32),
    )(feats, W, dinv)


def _tc_mid(acc2, xs2, dinv, b, Wn):
    def body(a_ref, xs_ref, dv_ref, b_ref, w_ref, o_ref):
        p = a_ref[...] + xs_ref[...]
        dv = dv_ref[...]
        x = jnp.concatenate([p[0], p[1]], axis=1) * dv + b_ref[...]
        o_ref[...] = jnp.dot(x, w_ref[...],
                             preferred_element_type=jnp.float32) * dv

    return pl.pallas_call(
        body,
        grid=(NC, N // NBLK),
        in_specs=[
            pl.BlockSpec((NC, NBLK, DH), lambda c, i: (0, i, 0)),
            pl.BlockSpec((NC, NBLK, DH), lambda c, i: (0, i, 0)),
            pl.BlockSpec((NBLK, 1), lambda c, i: (i, 0)),
            pl.BlockSpec((1, D), lambda c, i: (0, 0)),
            pl.BlockSpec((D, DH), lambda c, i: (0, c)),
        ],
        out_specs=pl.BlockSpec((None, NBLK, DH), lambda c, i: (c, i, 0)),
        out_shape=jax.ShapeDtypeStruct((NC, N, DH), jnp.float32),
    )(acc2, xs2, dinv, b, Wn)


def _tc_last(acc2, xs2, dinv, b):
    def body(a_ref, xs_ref, dv_ref, b_ref, o_ref):
        p = a_ref[...] + xs_ref[...]
        o_ref[...] = (jnp.concatenate([p[0], p[1]], axis=1) * dv_ref[...]
                      + b_ref[...])

    return pl.pallas_call(
        body,
        grid=(N // NBLK,),
        in_specs=[
            pl.BlockSpec((NC, NBLK, DH), lambda i: (0, i, 0)),
            pl.BlockSpec((NC, NBLK, DH), lambda i: (0, i, 0)),
            pl.BlockSpec((NBLK, 1), lambda i: (i, 0)),
            pl.BlockSpec((1, D), lambda i: (0, 0)),
        ],
        out_specs=pl.BlockSpec((NBLK, D), lambda i: (i, 0)),
        out_shape=jax.ShapeDtypeStruct((N, D), jnp.float32),
    )(acc2, xs2, dinv, b)


def kernel(feats, edges, W0, b0, W1, b1, W2, b2, W3, b3, W4, b4, W5, b5):
    Ws = (W0, W1, W2, W3, W4, W5)
    bs = (b0.reshape(1, D), b1.reshape(1, D), b2.reshape(1, D),
          b3.reshape(1, D), b4.reshape(1, D), b5.reshape(1, D))
    src = edges[0]
    dst = edges[1]
    pad = EPAD - E
    srcp = jnp.concatenate([src, jnp.zeros((pad,), jnp.int32)]).reshape(NS, NB, B)
    dst_flat = jnp.concatenate([dst, jnp.full((pad,), N, jnp.int32)])
    dstp = dst_flat.reshape(NS, NB, B)
    dstd = dst_flat.reshape(NC * NS, DEG_PT)
    iid = jnp.arange(DEG_R, dtype=jnp.int32).reshape(5, 128)

    degp = _sc_deg(dstd, iid)
    dinv_row = _tc_dinv(degp.reshape(NC, DEG_R * 16))
    dinv = dinv_row.reshape(-1, 1)[:N]

    xs = _tc_first(feats, W0, dinv)
    for l in range(6):
        acc = _sc_agg(xs, srcp, dstp)
        if l < 5:
            xs = _tc_mid(acc, xs, dinv, bs[l], Ws[l + 1])
    return _tc_last(acc, xs, dinv, bs[5])


# speedup vs baseline: 5.5092x; 5.5092x over previous
import functools

import jax
import jax.numpy as jnp
from jax import lax
from jax.experimental import pallas as pl
from jax.experimental.pallas import tpu as pltpu
from jax.experimental.pallas import tpu_sc as plsc

N = 10000
E = 160000
D = 256
NC = 2
NS = 16
DH = D // NC
B = 64
NB = 160
CH = 16
NCH = NB // CH
EPT = NB * B
EPAD = NS * EPT
N_ACC = 10240
ZSTRIPE = N_ACC // NS
WSTRIPE = 624
DEG_PT = EPAD // (NC * NS)
NPAD = 10240

_MESH = dict(core_axis_name="c", subcore_axis_name="s")


def _sc_deg(dstd):
    @functools.partial(
        pl.kernel,
        out_type=jax.ShapeDtypeStruct((NC * NS * NPAD,), jnp.float32),
        mesh=plsc.VectorSubcoreMesh(**_MESH),
        compiler_params=pltpu.CompilerParams(needs_layout_passes=False),
        scratch_types=[
            pltpu.VMEM((DEG_PT,), jnp.int32),
            pltpu.VMEM((NPAD,), jnp.float32),
        ],
    )
    def k(dstd_hbm, degp_hbm, dstv, degloc):
        c = lax.axis_index("c")
        s = lax.axis_index("s")
        w = s * NC + c
        pltpu.sync_copy(dstd_hbm.at[pl.ds(w * DEG_PT, DEG_PT)], dstv)
        z16 = jnp.zeros((16,), jnp.float32)
        for r in range(NPAD // 16):
            degloc[pl.ds(r * 16, 16)] = z16
        ones = jnp.ones((16,), jnp.float32)

        def it(i, carry):
            d = dstv[pl.ds(i * 16, 16)]
            plsc.addupdate_scatter(degloc, [d], ones)
            return carry

        lax.fori_loop(0, DEG_PT // 16, it, 0)
        pltpu.sync_copy(degloc, degp_hbm.at[pl.ds(w * NPAD, NPAD)])

    return k(dstd)


def _sc_agg(xs2, srcp, dstp):
    @functools.partial(
        pl.kernel,
        out_type=jax.ShapeDtypeStruct((NC, N, DH), jnp.float32),
        mesh=plsc.VectorSubcoreMesh(**_MESH),
        compiler_params=pltpu.CompilerParams(needs_layout_passes=False),
        scratch_types=[
            pltpu.VMEM((CH, B), jnp.int32),
            pltpu.VMEM((CH, B), jnp.int32),
            pltpu.VMEM((2, B, DH), jnp.float32),
            pltpu.VMEM_SHARED((N_ACC, DH), jnp.float32),
            pltpu.SemaphoreType.DMA,
            pltpu.SemaphoreType.DMA,
        ],
    )
    def k(xs_hbm, src_hbm, dst_hbm, acc_hbm, srcv, dstv, rows, acc_sh, sem0, sem1):
        c = lax.axis_index("c")
        s = lax.axis_index("s")
        z16 = jnp.zeros((16,), jnp.float32)
        for r in range(B):
            for q in range(DH // 16):
                rows[0, r, pl.ds(q * 16, 16)] = z16
        base = s * ZSTRIPE
        for q in range(ZSTRIPE // B):
            pltpu.sync_copy(rows.at[0], acc_sh.at[pl.ds(base + q * B, B)])
        plsc.subcore_barrier()

        xs_c = xs_hbm.at[c]

        def chunk(ch, carry):
            pltpu.sync_copy(src_hbm.at[s].at[pl.ds(ch * CH, CH)], srcv)
            pltpu.sync_copy(dst_hbm.at[s].at[pl.ds(ch * CH, CH)], dstv)

            def blk(jj, carry2):
                j0 = jj * 2
                j1 = j0 + 1
                cp0 = pltpu.async_copy(xs_c.at[srcv.at[j0]], rows.at[0], sem0)
                cp1 = pltpu.async_copy(xs_c.at[srcv.at[j1]], rows.at[1], sem1)
                cp0.wait()
                pltpu.sync_copy(rows.at[0], acc_sh.at[dstv.at[j0]], add=True)
                cp1.wait()
                pltpu.sync_copy(rows.at[1], acc_sh.at[dstv.at[j1]], add=True)
                return carry2

            lax.fori_loop(0, CH // 2, blk, 0)
            return carry

        lax.fori_loop(0, NCH, chunk, 0)
        plsc.subcore_barrier()
        wb = s * WSTRIPE
        pltpu.sync_copy(acc_sh.at[pl.ds(wb, WSTRIPE)],
                        acc_hbm.at[c].at[pl.ds(wb, WSTRIPE)])

        @pl.when(s == NS - 1)
        def _tail():
            t0 = NS * WSTRIPE
            pltpu.sync_copy(acc_sh.at[pl.ds(t0, N - t0)],
                            acc_hbm.at[c].at[pl.ds(t0, N - t0)])

    return k(xs2, srcp, dstp)


NBLK = 1000


def _tc_dinv(degp):
    def body(d_ref, o_ref):
        d = d_ref[...]
        o_ref[...] = lax.rsqrt(jnp.sum(d, axis=0, keepdims=True) + 1.0)

    blk = 1024
    return pl.pallas_call(
        body,
        grid=(NPAD // blk,),
        in_specs=[pl.BlockSpec((NC * NS, blk), lambda i: (0, i))],
        out_specs=pl.BlockSpec((1, blk), lambda i: (0, i)),
        out_shape=jax.ShapeDtypeStruct((1, NPAD), jnp.float32),
    )(degp)


def _tc_first(feats, W, dinv):
    def body(x_ref, w_ref, dv_ref, o_ref):
        x = x_ref[...]
        o_ref[...] = jnp.dot(x, w_ref[...],
                             preferred_element_type=jnp.float32) * dv_ref[...]

    return pl.pallas_call(
        body,
        grid=(NC, N // NBLK),
        in_specs=[
            pl.BlockSpec((NBLK, D), lambda c, i: (i, 0)),
            pl.BlockSpec((D, DH), lambda c, i: (0, c)),
            pl.BlockSpec((NBLK, 1), lambda c, i: (i, 0)),
        ],
        out_specs=pl.BlockSpec((None, NBLK, DH), lambda c, i: (c, i, 0)),
        out_shape=jax.ShapeDtypeStruct((NC, N, DH), jnp.float32),
    )(feats, W, dinv)


def _tc_mid(acc2, xs2, dinv, b, Wn):
    def body(a_ref, xs_ref, dv_ref, b_ref, w_ref, o_ref):
        p = a_ref[...] + xs_ref[...]
        dv = dv_ref[...]
        x = jnp.concatenate([p[0], p[1]], axis=1) * dv + b_ref[...]
        o_ref[...] = jnp.dot(x, w_ref[...],
                             preferred_element_type=jnp.float32) * dv

    return pl.pallas_call(
        body,
        grid=(NC, N // NBLK),
        in_specs=[
            pl.BlockSpec((NC, NBLK, DH), lambda c, i: (0, i, 0)),
            pl.BlockSpec((NC, NBLK, DH), lambda c, i: (0, i, 0)),
            pl.BlockSpec((NBLK, 1), lambda c, i: (i, 0)),
            pl.BlockSpec((1, D), lambda c, i: (0, 0)),
            pl.BlockSpec((D, DH), lambda c, i: (0, c)),
        ],
        out_specs=pl.BlockSpec((None, NBLK, DH), lambda c, i: (c, i, 0)),
        out_shape=jax.ShapeDtypeStruct((NC, N, DH), jnp.float32),
    )(acc2, xs2, dinv, b, Wn)


def _tc_last(acc2, xs2, dinv, b):
    def body(a_ref, xs_ref, dv_ref, b_ref, o_ref):
        p = a_ref[...] + xs_ref[...]
        o_ref[...] = (jnp.concatenate([p[0], p[1]], axis=1) * dv_ref[...]
                      + b_ref[...])

    return pl.pallas_call(
        body,
        grid=(N // NBLK,),
        in_specs=[
            pl.BlockSpec((NC, NBLK, DH), lambda i: (0, i, 0)),
            pl.BlockSpec((NC, NBLK, DH), lambda i: (0, i, 0)),
            pl.BlockSpec((NBLK, 1), lambda i: (i, 0)),
            pl.BlockSpec((1, D), lambda i: (0, 0)),
        ],
        out_specs=pl.BlockSpec((NBLK, D), lambda i: (i, 0)),
        out_shape=jax.ShapeDtypeStruct((N, D), jnp.float32),
    )(acc2, xs2, dinv, b)


def kernel(feats, edges, W0, b0, W1, b1, W2, b2, W3, b3, W4, b4, W5, b5):
    Ws = (W0, W1, W2, W3, W4, W5)
    bs = (b0.reshape(1, D), b1.reshape(1, D), b2.reshape(1, D),
          b3.reshape(1, D), b4.reshape(1, D), b5.reshape(1, D))
    src = edges[0]
    dst = edges[1]
    pad = EPAD - E
    srcp = jnp.concatenate([src, jnp.zeros((pad,), jnp.int32)]).reshape(NS, NB, B)
    dst_flat = jnp.concatenate([dst, jnp.full((pad,), N, jnp.int32)])
    dstp = dst_flat.reshape(NS, NB, B)
    degp = _sc_deg(dst_flat)
    dinv_row = _tc_dinv(degp.reshape(NC * NS, NPAD))
    dinv = dinv_row.reshape(-1, 1)[:N]

    xs = _tc_first(feats, W0, dinv)
    for l in range(6):
        acc = _sc_agg(xs, srcp, dstp)
        if l < 5:
            xs = _tc_mid(acc, xs, dinv, bs[l], Ws[l + 1])
    return _tc_last(acc, xs, dinv, bs[5])


# speedup vs baseline: 5.6160x; 1.0194x over previous
import functools

import jax
import jax.numpy as jnp
from jax import lax
from jax.experimental import pallas as pl
from jax.experimental.pallas import tpu as pltpu
from jax.experimental.pallas import tpu_sc as plsc

N = 10000
E = 160000
D = 256
NC = 2
NS = 16
DH = D // NC
B = 64
NB = 160
CH = 16
NCH = NB // CH
EPT = NB * B
EPAD = NS * EPT
N_ACC = 10240
ZSTRIPE = N_ACC // NS
WSTRIPE = 624
DEG_PT = EPAD // (NC * NS)
NPAD = 10240

_MESH = dict(core_axis_name="c", subcore_axis_name="s")


def _sc_deg(dstd):
    @functools.partial(
        pl.kernel,
        out_type=jax.ShapeDtypeStruct((NC * NS * NPAD,), jnp.float32),
        mesh=plsc.VectorSubcoreMesh(**_MESH),
        compiler_params=pltpu.CompilerParams(needs_layout_passes=False),
        scratch_types=[
            pltpu.VMEM((DEG_PT,), jnp.int32),
            pltpu.VMEM((NPAD,), jnp.float32),
        ],
    )
    def k(dstd_hbm, degp_hbm, dstv, degloc):
        c = lax.axis_index("c")
        s = lax.axis_index("s")
        w = s * NC + c
        pltpu.sync_copy(dstd_hbm.at[pl.ds(w * DEG_PT, DEG_PT)], dstv)
        z16 = jnp.zeros((16,), jnp.float32)
        for r in range(NPAD // 16):
            degloc[pl.ds(r * 16, 16)] = z16
        ones = jnp.ones((16,), jnp.float32)

        def it(i, carry):
            d = dstv[pl.ds(i * 16, 16)]
            plsc.addupdate_scatter(degloc, [d], ones)
            return carry

        lax.fori_loop(0, DEG_PT // 16, it, 0)
        pltpu.sync_copy(degloc, degp_hbm.at[pl.ds(w * NPAD, NPAD)])

    return k(dstd)


def _sc_agg(xs2, srcp, dstp):
    @functools.partial(
        pl.kernel,
        out_type=jax.ShapeDtypeStruct((NC, N, DH), jnp.float32),
        mesh=plsc.VectorSubcoreMesh(**_MESH),
        compiler_params=pltpu.CompilerParams(needs_layout_passes=False),
        scratch_types=[
            pltpu.VMEM((2, CH, B), jnp.int32),
            pltpu.VMEM((2, CH, B), jnp.int32),
            pltpu.VMEM((2, B, DH), jnp.float32),
            pltpu.VMEM_SHARED((N_ACC, DH), jnp.float32),
            pltpu.SemaphoreType.DMA,
            pltpu.SemaphoreType.DMA,
            pltpu.SemaphoreType.DMA,
        ],
    )
    def k(xs_hbm, src_hbm, dst_hbm, acc_hbm, srcv, dstv, rows, acc_sh,
          gsem, ssem, isem):
        c = lax.axis_index("c")
        s = lax.axis_index("s")
        z16 = jnp.zeros((16,), jnp.float32)
        for r in range(B):
            for q in range(DH // 16):
                rows[0, r, pl.ds(q * 16, 16)] = z16
        base = s * ZSTRIPE
        for q in range(ZSTRIPE // B):
            pltpu.sync_copy(rows.at[0], acc_sh.at[pl.ds(base + q * B, B)])
        plsc.subcore_barrier()

        xs_c = xs_hbm.at[c]
        src_t = src_hbm.at[s]
        dst_t = dst_hbm.at[s]

        pltpu.async_copy(src_t.at[pl.ds(0, CH)], srcv.at[0], isem)
        pltpu.async_copy(dst_t.at[pl.ds(0, CH)], dstv.at[0], isem)

        def chunk(ch, carry):
            sl = lax.rem(ch, 2)
            pltpu.make_async_copy(src_t.at[pl.ds(0, CH)], srcv.at[sl], isem).wait()
            pltpu.make_async_copy(dst_t.at[pl.ds(0, CH)], dstv.at[sl], isem).wait()

            @pl.when(ch < NCH - 1)
            def _prefetch():
                nsl = lax.rem(ch + 1, 2)
                off = (ch + 1) * CH
                pltpu.async_copy(src_t.at[pl.ds(off, CH)], srcv.at[nsl], isem)
                pltpu.async_copy(dst_t.at[pl.ds(off, CH)], dstv.at[nsl], isem)

            sv = srcv.at[sl]
            dv = dstv.at[sl]
            g = pltpu.async_copy(xs_c.at[sv.at[0]], rows.at[0], gsem)
            sc_prev = None
            for b in range(CH):
                g.wait()
                sc = pltpu.async_copy(rows.at[b % 2], acc_sh.at[dv.at[b]],
                                      ssem, add=True)
                if b + 1 < CH:
                    if sc_prev is not None:
                        sc_prev.wait()
                    g = pltpu.async_copy(xs_c.at[sv.at[b + 1]],
                                         rows.at[(b + 1) % 2], gsem)
                sc_prev = sc
            sc_prev.wait()
            return carry

        lax.fori_loop(0, NCH, chunk, 0)
        plsc.subcore_barrier()
        wb = s * WSTRIPE
        pltpu.sync_copy(acc_sh.at[pl.ds(wb, WSTRIPE)],
                        acc_hbm.at[c].at[pl.ds(wb, WSTRIPE)])

        @pl.when(s == NS - 1)
        def _tail():
            t0 = NS * WSTRIPE
            pltpu.sync_copy(acc_sh.at[pl.ds(t0, N - t0)],
                            acc_hbm.at[c].at[pl.ds(t0, N - t0)])

    return k(xs2, srcp, dstp)


NBLK = 1000


def _tc_dinv(degp):
    def body(d_ref, o_ref):
        d = d_ref[...]
        o_ref[...] = lax.rsqrt(jnp.sum(d, axis=0, keepdims=True) + 1.0)

    blk = 1024
    return pl.pallas_call(
        body,
        grid=(NPAD // blk,),
        in_specs=[pl.BlockSpec((NC * NS, blk), lambda i: (0, i))],
        out_specs=pl.BlockSpec((1, blk), lambda i: (0, i)),
        out_shape=jax.ShapeDtypeStruct((1, NPAD), jnp.float32),
    )(degp)


def _tc_first(feats, W, dinv):
    def body(x_ref, w_ref, dv_ref, o_ref):
        x = x_ref[...]
        o_ref[...] = jnp.dot(x, w_ref[...],
                             preferred_element_type=jnp.float32) * dv_ref[...]

    return pl.pallas_call(
        body,
        grid=(NC, N // NBLK),
        in_specs=[
            pl.BlockSpec((NBLK, D), lambda c, i: (i, 0)),
            pl.BlockSpec((D, DH), lambda c, i: (0, c)),
            pl.BlockSpec((NBLK, 1), lambda c, i: (i, 0)),
        ],
        out_specs=pl.BlockSpec((None, NBLK, DH), lambda c, i: (c, i, 0)),
        out_shape=jax.ShapeDtypeStruct((NC, N, DH), jnp.float32),
    )(feats, W, dinv)


def _tc_mid(acc2, xs2, dinv, b, Wn):
    def body(a_ref, xs_ref, dv_ref, b_ref, w_ref, o_ref):
        p = a_ref[...] + xs_ref[...]
        dv = dv_ref[...]
        x = jnp.concatenate([p[0], p[1]], axis=1) * dv + b_ref[...]
        o_ref[...] = jnp.dot(x, w_ref[...],
                             preferred_element_type=jnp.float32) * dv

    return pl.pallas_call(
        body,
        grid=(NC, N // NBLK),
        in_specs=[
            pl.BlockSpec((NC, NBLK, DH), lambda c, i: (0, i, 0)),
            pl.BlockSpec((NC, NBLK, DH), lambda c, i: (0, i, 0)),
            pl.BlockSpec((NBLK, 1), lambda c, i: (i, 0)),
            pl.BlockSpec((1, D), lambda c, i: (0, 0)),
            pl.BlockSpec((D, DH), lambda c, i: (0, c)),
        ],
        out_specs=pl.BlockSpec((None, NBLK, DH), lambda c, i: (c, i, 0)),
        out_shape=jax.ShapeDtypeStruct((NC, N, DH), jnp.float32),
    )(acc2, xs2, dinv, b, Wn)


def _tc_last(acc2, xs2, dinv, b):
    def body(a_ref, xs_ref, dv_ref, b_ref, o_ref):
        p = a_ref[...] + xs_ref[...]
        o_ref[...] = (jnp.concatenate([p[0], p[1]], axis=1) * dv_ref[...]
                      + b_ref[...])

    return pl.pallas_call(
        body,
        grid=(N // NBLK,),
        in_specs=[
            pl.BlockSpec((NC, NBLK, DH), lambda i: (0, i, 0)),
            pl.BlockSpec((NC, NBLK, DH), lambda i: (0, i, 0)),
            pl.BlockSpec((NBLK, 1), lambda i: (i, 0)),
            pl.BlockSpec((1, D), lambda i: (0, 0)),
        ],
        out_specs=pl.BlockSpec((NBLK, D), lambda i: (i, 0)),
        out_shape=jax.ShapeDtypeStruct((N, D), jnp.float32),
    )(acc2, xs2, dinv, b)


def kernel(feats, edges, W0, b0, W1, b1, W2, b2, W3, b3, W4, b4, W5, b5):
    Ws = (W0, W1, W2, W3, W4, W5)
    bs = (b0.reshape(1, D), b1.reshape(1, D), b2.reshape(1, D),
          b3.reshape(1, D), b4.reshape(1, D), b5.reshape(1, D))
    src = edges[0]
    dst = edges[1]
    pad = EPAD - E
    srcp = jnp.concatenate([src, jnp.zeros((pad,), jnp.int32)]).reshape(NS, NB, B)
    dst_flat = jnp.concatenate([dst, jnp.full((pad,), N, jnp.int32)])
    dstp = dst_flat.reshape(NS, NB, B)
    degp = _sc_deg(dst_flat)
    dinv_row = _tc_dinv(degp.reshape(NC * NS, NPAD))
    dinv = dinv_row.reshape(-1, 1)[:N]

    xs = _tc_first(feats, W0, dinv)
    for l in range(6):
        acc = _sc_agg(xs, srcp, dstp)
        if l < 5:
            xs = _tc_mid(acc, xs, dinv, bs[l], Ws[l + 1])
    return _tc_last(acc, xs, dinv, bs[5])


# speedup vs baseline: 6.4769x; 1.1533x over previous
import functools

import jax
import jax.numpy as jnp
from jax import lax
from jax.experimental import pallas as pl
from jax.experimental.pallas import tpu as pltpu
from jax.experimental.pallas import tpu_sc as plsc

N = 10000
E = 160000
D = 256
NC = 2
NS = 16
DH = D // NC
B = 40
NB = 256
CH = 32
NCH = NB // CH
NBUF = 4
EPT = NB * B
EPAD = NS * EPT
N_ACC = 10240
ZSTRIPE = N_ACC // NS
WSTRIPE = 624
DEG_PT = EPAD // (NC * NS)
NPAD = 10240

_MESH = dict(core_axis_name="c", subcore_axis_name="s")


def _sc_deg(dstd):
    @functools.partial(
        pl.kernel,
        out_type=jax.ShapeDtypeStruct((NC * NS * NPAD,), jnp.float32),
        mesh=plsc.VectorSubcoreMesh(**_MESH),
        compiler_params=pltpu.CompilerParams(needs_layout_passes=False),
        scratch_types=[
            pltpu.VMEM((DEG_PT,), jnp.int32),
            pltpu.VMEM((NPAD,), jnp.float32),
        ],
    )
    def k(dstd_hbm, degp_hbm, dstv, degloc):
        c = lax.axis_index("c")
        s = lax.axis_index("s")
        w = s * NC + c
        pltpu.sync_copy(dstd_hbm.at[pl.ds(w * DEG_PT, DEG_PT)], dstv)
        z16 = jnp.zeros((16,), jnp.float32)
        for r in range(NPAD // 16):
            degloc[pl.ds(r * 16, 16)] = z16
        ones = jnp.ones((16,), jnp.float32)

        def it(i, carry):
            d = dstv[pl.ds(i * 16, 16)]
            plsc.addupdate_scatter(degloc, [d], ones)
            return carry

        lax.fori_loop(0, DEG_PT // 16, it, 0)
        pltpu.sync_copy(degloc, degp_hbm.at[pl.ds(w * NPAD, NPAD)])

    return k(dstd)


def _sc_agg(xs2, srcp, dstp):
    @functools.partial(
        pl.kernel,
        out_type=jax.ShapeDtypeStruct((NC, N, DH), jnp.float32),
        mesh=plsc.VectorSubcoreMesh(**_MESH),
        compiler_params=pltpu.CompilerParams(needs_layout_passes=False),
        scratch_types=[
            pltpu.VMEM((2, CH, B), jnp.int32),
            pltpu.VMEM((2, CH, B), jnp.int32),
            pltpu.VMEM((NBUF, B, DH), jnp.float32),
            pltpu.VMEM_SHARED((N_ACC, DH), jnp.float32),
            pltpu.SemaphoreType.DMA,
            pltpu.SemaphoreType.DMA,
            pltpu.SemaphoreType.DMA,
        ],
    )
    def k(xs_hbm, src_hbm, dst_hbm, acc_hbm, srcv, dstv, rows, acc_sh,
          gsem, ssem, isem):
        c = lax.axis_index("c")
        s = lax.axis_index("s")
        z16 = jnp.zeros((16,), jnp.float32)
        for r in range(B):
            for q in range(DH // 16):
                rows[0, r, pl.ds(q * 16, 16)] = z16
        base = s * ZSTRIPE
        for q in range(ZSTRIPE // B):
            pltpu.sync_copy(rows.at[0], acc_sh.at[pl.ds(base + q * B, B)])
        plsc.subcore_barrier()
        assert ZSTRIPE % B == 0

        xs_c = xs_hbm.at[c]
        src_t = src_hbm.at[s]
        dst_t = dst_hbm.at[s]

        pltpu.async_copy(src_t.at[pl.ds(0, CH)], srcv.at[0], isem)
        pltpu.async_copy(dst_t.at[pl.ds(0, CH)], dstv.at[0], isem)

        def chunk(ch, carry):
            sl = lax.rem(ch, 2)
            pltpu.make_async_copy(src_t.at[pl.ds(0, CH)], srcv.at[sl], isem).wait()
            pltpu.make_async_copy(dst_t.at[pl.ds(0, CH)], dstv.at[sl], isem).wait()

            @pl.when(ch < NCH - 1)
            def _prefetch():
                nsl = lax.rem(ch + 1, 2)
                off = (ch + 1) * CH
                pltpu.async_copy(src_t.at[pl.ds(off, CH)], srcv.at[nsl], isem)
                pltpu.async_copy(dst_t.at[pl.ds(off, CH)], dstv.at[nsl], isem)

            sv = srcv.at[sl]
            dv = dstv.at[sl]
            gd = {}
            sd = {}
            for b in range(NBUF - 1):
                gd[b] = pltpu.async_copy(xs_c.at[sv.at[b]],
                                         rows.at[b % NBUF], gsem)
            for b in range(CH):
                gd[b].wait()
                sd[b] = pltpu.async_copy(rows.at[b % NBUF],
                                         acc_sh.at[dv.at[b]], ssem, add=True)
                nb = b + NBUF - 1
                if nb < CH:
                    if b >= 1:
                        sd[b - 1].wait()
                    gd[nb] = pltpu.async_copy(xs_c.at[sv.at[nb]],
                                              rows.at[nb % NBUF], gsem)
            for b in range(CH - NBUF, CH):
                sd[b].wait()
            return carry

        lax.fori_loop(0, NCH, chunk, 0)
        plsc.subcore_barrier()
        wb = s * WSTRIPE
        pltpu.sync_copy(acc_sh.at[pl.ds(wb, WSTRIPE)],
                        acc_hbm.at[c].at[pl.ds(wb, WSTRIPE)])

        @pl.when(s == NS - 1)
        def _tail():
            t0 = NS * WSTRIPE
            pltpu.sync_copy(acc_sh.at[pl.ds(t0, N - t0)],
                            acc_hbm.at[c].at[pl.ds(t0, N - t0)])

    return k(xs2, srcp, dstp)


NBLK = 1000


def _tc_dinv(degp):
    def body(d_ref, o_ref):
        d = d_ref[...]
        o_ref[...] = lax.rsqrt(jnp.sum(d, axis=0, keepdims=True) + 1.0)

    blk = 1024
    return pl.pallas_call(
        body,
        grid=(NPAD // blk,),
        in_specs=[pl.BlockSpec((NC * NS, blk), lambda i: (0, i))],
        out_specs=pl.BlockSpec((1, blk), lambda i: (0, i)),
        out_shape=jax.ShapeDtypeStruct((1, NPAD), jnp.float32),
    )(degp)


def _tc_first(feats, W, dinv):
    def body(x_ref, w_ref, dv_ref, o_ref):
        x = x_ref[...]
        o_ref[...] = jnp.dot(x, w_ref[...],
                             preferred_element_type=jnp.float32) * dv_ref[...]

    return pl.pallas_call(
        body,
        grid=(NC, N // NBLK),
        in_specs=[
            pl.BlockSpec((NBLK, D), lambda c, i: (i, 0)),
            pl.BlockSpec((D, DH), lambda c, i: (0, c)),
            pl.BlockSpec((NBLK, 1), lambda c, i: (i, 0)),
        ],
        out_specs=pl.BlockSpec((None, NBLK, DH), lambda c, i: (c, i, 0)),
        out_shape=jax.ShapeDtypeStruct((NC, N, DH), jnp.float32),
    )(feats, W, dinv)


def _tc_mid(acc2, xs2, dinv, b, Wn):
    def body(a_ref, xs_ref, dv_ref, b_ref, w_ref, o_ref):
        p = a_ref[...] + xs_ref[...]
        dv = dv_ref[...]
        x = jnp.concatenate([p[0], p[1]], axis=1) * dv + b_ref[...]
        o_ref[...] = jnp.dot(x, w_ref[...],
                             preferred_element_type=jnp.float32) * dv

    return pl.pallas_call(
        body,
        grid=(NC, N // NBLK),
        in_specs=[
            pl.BlockSpec((NC, NBLK, DH), lambda c, i: (0, i, 0)),
            pl.BlockSpec((NC, NBLK, DH), lambda c, i: (0, i, 0)),
            pl.BlockSpec((NBLK, 1), lambda c, i: (i, 0)),
            pl.BlockSpec((1, D), lambda c, i: (0, 0)),
            pl.BlockSpec((D, DH), lambda c, i: (0, c)),
        ],
        out_specs=pl.BlockSpec((None, NBLK, DH), lambda c, i: (c, i, 0)),
        out_shape=jax.ShapeDtypeStruct((NC, N, DH), jnp.float32),
    )(acc2, xs2, dinv, b, Wn)


def _tc_last(acc2, xs2, dinv, b):
    def body(a_ref, xs_ref, dv_ref, b_ref, o_ref):
        p = a_ref[...] + xs_ref[...]
        o_ref[...] = (jnp.concatenate([p[0], p[1]], axis=1) * dv_ref[...]
                      + b_ref[...])

    return pl.pallas_call(
        body,
        grid=(N // NBLK,),
        in_specs=[
            pl.BlockSpec((NC, NBLK, DH), lambda i: (0, i, 0)),
            pl.BlockSpec((NC, NBLK, DH), lambda i: (0, i, 0)),
            pl.BlockSpec((NBLK, 1), lambda i: (i, 0)),
            pl.BlockSpec((1, D), lambda i: (0, 0)),
        ],
        out_specs=pl.BlockSpec((NBLK, D), lambda i: (i, 0)),
        out_shape=jax.ShapeDtypeStruct((N, D), jnp.float32),
    )(acc2, xs2, dinv, b)


def kernel(feats, edges, W0, b0, W1, b1, W2, b2, W3, b3, W4, b4, W5, b5):
    Ws = (W0, W1, W2, W3, W4, W5)
    bs = (b0.reshape(1, D), b1.reshape(1, D), b2.reshape(1, D),
          b3.reshape(1, D), b4.reshape(1, D), b5.reshape(1, D))
    src = edges[0]
    dst = edges[1]
    pad = EPAD - E
    srcp = jnp.concatenate([src, jnp.zeros((pad,), jnp.int32)]).reshape(NS, NB, B)
    dst_flat = jnp.concatenate([dst, jnp.full((pad,), N, jnp.int32)])
    dstp = dst_flat.reshape(NS, NB, B)
    degp = _sc_deg(dst_flat)
    dinv_row = _tc_dinv(degp.reshape(NC * NS, NPAD))
    dinv = dinv_row.reshape(-1, 1)[:N]

    xs = _tc_first(feats, W0, dinv)
    for l in range(6):
        acc = _sc_agg(xs, srcp, dstp)
        if l < 5:
            xs = _tc_mid(acc, xs, dinv, bs[l], Ws[l + 1])
    return _tc_last(acc, xs, dinv, bs[5])


# speedup vs baseline: 6.6330x; 1.0241x over previous
import functools

import jax
import jax.numpy as jnp
from jax import lax
from jax.experimental import pallas as pl
from jax.experimental.pallas import tpu as pltpu
from jax.experimental.pallas import tpu_sc as plsc

N = 10000
E = 160000
D = 256
NC = 2
NS = 16
DH = D // NC
B = 64
NB = 160
CH = 16
NCH = NB // CH
NBUF = 3
EPT = NB * B
EPAD = NS * EPT
N_ACC = 10112
ZSTRIPE = N_ACC // NS
WSTRIPE = 624
DEG_PT = EPAD // (NC * NS)
NPAD = 10240

_MESH = dict(core_axis_name="c", subcore_axis_name="s")


def _sc_deg(dstd):
    @functools.partial(
        pl.kernel,
        out_type=jax.ShapeDtypeStruct((NC * NS * NPAD,), jnp.float32),
        mesh=plsc.VectorSubcoreMesh(**_MESH),
        compiler_params=pltpu.CompilerParams(needs_layout_passes=False),
        scratch_types=[
            pltpu.VMEM((DEG_PT,), jnp.int32),
            pltpu.VMEM((NPAD,), jnp.float32),
        ],
    )
    def k(dstd_hbm, degp_hbm, dstv, degloc):
        c = lax.axis_index("c")
        s = lax.axis_index("s")
        w = s * NC + c
        pltpu.sync_copy(dstd_hbm.at[pl.ds(w * DEG_PT, DEG_PT)], dstv)
        z16 = jnp.zeros((16,), jnp.float32)
        for r in range(NPAD // 16):
            degloc[pl.ds(r * 16, 16)] = z16
        ones = jnp.ones((16,), jnp.float32)

        def it(i, carry):
            d = dstv[pl.ds(i * 16, 16)]
            plsc.addupdate_scatter(degloc, [d], ones)
            return carry

        lax.fori_loop(0, DEG_PT // 16, it, 0)
        pltpu.sync_copy(degloc, degp_hbm.at[pl.ds(w * NPAD, NPAD)])

    return k(dstd)


def _sc_agg(xs2, srcp, dstp):
    @functools.partial(
        pl.kernel,
        out_type=jax.ShapeDtypeStruct((NC, N, DH), jnp.float32),
        mesh=plsc.VectorSubcoreMesh(**_MESH),
        compiler_params=pltpu.CompilerParams(needs_layout_passes=False),
        scratch_types=[
            pltpu.VMEM((2, CH, B), jnp.int32),
            pltpu.VMEM((2, CH, B), jnp.int32),
            pltpu.VMEM((NBUF, B, DH), jnp.float32),
            pltpu.VMEM_SHARED((N_ACC, DH), jnp.float32),
            pltpu.SemaphoreType.DMA,
            pltpu.SemaphoreType.DMA,
            pltpu.SemaphoreType.DMA,
        ],
    )
    def k(xs_hbm, src_hbm, dst_hbm, acc_hbm, srcv, dstv, rows, acc_sh,
          gsem, ssem, isem):
        c = lax.axis_index("c")
        s = lax.axis_index("s")
        z16 = jnp.zeros((16,), jnp.float32)
        for r in range(B):
            for q in range(DH // 16):
                rows[0, r, pl.ds(q * 16, 16)] = z16
        base = s * ZSTRIPE
        for q in range(ZSTRIPE // B):
            pltpu.sync_copy(rows.at[0], acc_sh.at[pl.ds(base + q * B, B)])
        zrem = ZSTRIPE % B
        if zrem:
            pltpu.sync_copy(rows.at[0, pl.ds(0, zrem)],
                            acc_sh.at[pl.ds(base + ZSTRIPE - zrem, zrem)])
        plsc.subcore_barrier()

        xs_c = xs_hbm.at[c]
        src_t = src_hbm.at[s]
        dst_t = dst_hbm.at[s]

        pltpu.async_copy(src_t.at[pl.ds(0, CH)], srcv.at[0], isem)
        pltpu.async_copy(dst_t.at[pl.ds(0, CH)], dstv.at[0], isem)

        def chunk(ch, carry):
            sl = lax.rem(ch, 2)
            pltpu.make_async_copy(src_t.at[pl.ds(0, CH)], srcv.at[sl], isem).wait()
            pltpu.make_async_copy(dst_t.at[pl.ds(0, CH)], dstv.at[sl], isem).wait()

            @pl.when(ch < NCH - 1)
            def _prefetch():
                nsl = lax.rem(ch + 1, 2)
                off = (ch + 1) * CH
                pltpu.async_copy(src_t.at[pl.ds(off, CH)], srcv.at[nsl], isem)
                pltpu.async_copy(dst_t.at[pl.ds(off, CH)], dstv.at[nsl], isem)

            sv = srcv.at[sl]
            dv = dstv.at[sl]
            gd = {}
            sd = {}
            for b in range(NBUF - 1):
                gd[b] = pltpu.async_copy(xs_c.at[sv.at[b]],
                                         rows.at[b % NBUF], gsem)
            for b in range(CH):
                gd[b].wait()
                sd[b] = pltpu.async_copy(rows.at[b % NBUF],
                                         acc_sh.at[dv.at[b]], ssem, add=True)
                nb = b + NBUF - 1
                if nb < CH:
                    if b >= 1:
                        sd[b - 1].wait()
                    gd[nb] = pltpu.async_copy(xs_c.at[sv.at[nb]],
                                              rows.at[nb % NBUF], gsem)
            for b in range(CH - NBUF, CH):
                sd[b].wait()
            return carry

        lax.fori_loop(0, NCH, chunk, 0)
        plsc.subcore_barrier()
        wb = s * WSTRIPE
        pltpu.sync_copy(acc_sh.at[pl.ds(wb, WSTRIPE)],
                        acc_hbm.at[c].at[pl.ds(wb, WSTRIPE)])

        @pl.when(s == NS - 1)
        def _tail():
            t0 = NS * WSTRIPE
            pltpu.sync_copy(acc_sh.at[pl.ds(t0, N - t0)],
                            acc_hbm.at[c].at[pl.ds(t0, N - t0)])

    return k(xs2, srcp, dstp)


NBLK = 1000


def _tc_dinv(degp):
    def body(d_ref, o_ref):
        d = d_ref[...]
        o_ref[...] = lax.rsqrt(jnp.sum(d, axis=0, keepdims=True) + 1.0)

    blk = 1024
    return pl.pallas_call(
        body,
        grid=(NPAD // blk,),
        in_specs=[pl.BlockSpec((NC * NS, blk), lambda i: (0, i))],
        out_specs=pl.BlockSpec((1, blk), lambda i: (0, i)),
        out_shape=jax.ShapeDtypeStruct((1, NPAD), jnp.float32),
    )(degp)


def _tc_first(feats, W, dinv):
    def body(x_ref, w_ref, dv_ref, o_ref):
        x = x_ref[...]
        o_ref[...] = jnp.dot(x, w_ref[...],
                             preferred_element_type=jnp.float32) * dv_ref[...]

    return pl.pallas_call(
        body,
        grid=(NC, N // NBLK),
        in_specs=[
            pl.BlockSpec((NBLK, D), lambda c, i: (i, 0)),
            pl.BlockSpec((D, DH), lambda c, i: (0, c)),
            pl.BlockSpec((NBLK, 1), lambda c, i: (i, 0)),
        ],
        out_specs=pl.BlockSpec((None, NBLK, DH), lambda c, i: (c, i, 0)),
        out_shape=jax.ShapeDtypeStruct((NC, N, DH), jnp.float32),
    )(feats, W, dinv)


def _tc_mid(acc2, xs2, dinv, b, Wn):
    def body(a_ref, xs_ref, dv_ref, b_ref, w_ref, o_ref):
        p = a_ref[...] + xs_ref[...]
        dv = dv_ref[...]
        x = jnp.concatenate([p[0], p[1]], axis=1) * dv + b_ref[...]
        o_ref[...] = jnp.dot(x, w_ref[...],
                             preferred_element_type=jnp.float32) * dv

    return pl.pallas_call(
        body,
        grid=(NC, N // NBLK),
        in_specs=[
            pl.BlockSpec((NC, NBLK, DH), lambda c, i: (0, i, 0)),
            pl.BlockSpec((NC, NBLK, DH), lambda c, i: (0, i, 0)),
            pl.BlockSpec((NBLK, 1), lambda c, i: (i, 0)),
            pl.BlockSpec((1, D), lambda c, i: (0, 0)),
            pl.BlockSpec((D, DH), lambda c, i: (0, c)),
        ],
        out_specs=pl.BlockSpec((None, NBLK, DH), lambda c, i: (c, i, 0)),
        out_shape=jax.ShapeDtypeStruct((NC, N, DH), jnp.float32),
    )(acc2, xs2, dinv, b, Wn)


def _tc_last(acc2, xs2, dinv, b):
    def body(a_ref, xs_ref, dv_ref, b_ref, o_ref):
        p = a_ref[...] + xs_ref[...]
        o_ref[...] = (jnp.concatenate([p[0], p[1]], axis=1) * dv_ref[...]
                      + b_ref[...])

    return pl.pallas_call(
        body,
        grid=(N // NBLK,),
        in_specs=[
            pl.BlockSpec((NC, NBLK, DH), lambda i: (0, i, 0)),
            pl.BlockSpec((NC, NBLK, DH), lambda i: (0, i, 0)),
            pl.BlockSpec((NBLK, 1), lambda i: (i, 0)),
            pl.BlockSpec((1, D), lambda i: (0, 0)),
        ],
        out_specs=pl.BlockSpec((NBLK, D), lambda i: (i, 0)),
        out_shape=jax.ShapeDtypeStruct((N, D), jnp.float32),
    )(acc2, xs2, dinv, b)


def kernel(feats, edges, W0, b0, W1, b1, W2, b2, W3, b3, W4, b4, W5, b5):
    Ws = (W0, W1, W2, W3, W4, W5)
    bs = (b0.reshape(1, D), b1.reshape(1, D), b2.reshape(1, D),
          b3.reshape(1, D), b4.reshape(1, D), b5.reshape(1, D))
    src = edges[0]
    dst = edges[1]
    pad = EPAD - E
    srcp = jnp.concatenate([src, jnp.zeros((pad,), jnp.int32)]).reshape(NS, NB, B)
    dst_flat = jnp.concatenate([dst, jnp.full((pad,), N, jnp.int32)])
    dstp = dst_flat.reshape(NS, NB, B)
    degp = _sc_deg(dst_flat)
    dinv_row = _tc_dinv(degp.reshape(NC * NS, NPAD))
    dinv = dinv_row.reshape(-1, 1)[:N]

    xs = _tc_first(feats, W0, dinv)
    for l in range(6):
        acc = _sc_agg(xs, srcp, dstp)
        if l < 5:
            xs = _tc_mid(acc, xs, dinv, bs[l], Ws[l + 1])
    return _tc_last(acc, xs, dinv, bs[5])
